# baseline (speedup 1.0000x reference)
import functools

import jax
import jax.numpy as jnp
from jax import lax
from jax.experimental import pallas as pl
from jax.experimental.pallas import tpu as pltpu
from jax.experimental.pallas import tpu_sc as plsc

NC = 2
NS = 16
LANES = 16
NW = NC * NS
CHUNK = 128
DEG_W = 16


def _sc_degree(dst_r, n_pad):
    cpw = dst_r.shape[1]
    stripe = n_pad // NS
    mesh = plsc.VectorSubcoreMesh(core_axis_name="c", subcore_axis_name="s")

    @functools.partial(
        pl.kernel,
        out_type=jax.ShapeDtypeStruct((NC, n_pad), jnp.float32),
        mesh=mesh,
        scratch_types=[
            pltpu.VMEM((cpw, CHUNK), jnp.int32),
            pltpu.VMEM((CHUNK,), jnp.float32),
            pltpu.VMEM((stripe,), jnp.float32),
            pltpu.VMEM_SHARED((n_pad,), jnp.float32),
        ],
    )
    def deg_kernel(dst_hbm, out_hbm, dst_v, ones_v, zero_v, hist_sh):
        c = lax.axis_index("c")
        s = lax.axis_index("s")
        wid = s * NC + c

        for r in range(CHUNK // LANES):
            ones_v[pl.ds(r * LANES, LANES)] = jnp.ones((LANES,), jnp.float32)

        def zfill(r, carry):
            zero_v[pl.ds(r * LANES, LANES)] = jnp.zeros((LANES,), jnp.float32)
            return carry

        lax.fori_loop(0, stripe // LANES, zfill, 0)
        pltpu.sync_copy(zero_v, hist_sh.at[pl.ds(s * stripe, stripe)])
        plsc.subcore_barrier()

        pltpu.sync_copy(dst_hbm.at[wid], dst_v)

        def body(j, carry):
            pltpu.sync_copy(ones_v, hist_sh.at[dst_v.at[j]], add=True)
            return carry

        lax.fori_loop(0, cpw, body, 0)
        plsc.subcore_barrier()
        pltpu.sync_copy(hist_sh.at[pl.ds(s * stripe, stripe)],
                        out_hbm.at[c, pl.ds(s * stripe, stripe)])

    return deg_kernel(dst_r)


def _sc_aggregate(q, src_r, dst_r, n_pad, t0, t1):
    d = q.shape[1]
    cpw = src_r.shape[1]
    assert max(t0, t1) <= cpw
    stripe = n_pad // NS
    mesh = plsc.VectorSubcoreMesh(core_axis_name="c", subcore_axis_name="s")

    @functools.partial(
        pl.kernel,
        out_type=jax.ShapeDtypeStruct((NC, n_pad, d), jnp.float32),
        mesh=mesh,
        scratch_types=[
            pltpu.VMEM((cpw, CHUNK), jnp.int32),
            pltpu.VMEM((cpw, CHUNK), jnp.int32),
            pltpu.VMEM((CHUNK, d), jnp.float32),
            pltpu.VMEM_SHARED((n_pad, d), jnp.float32),
            pltpu.SemaphoreType.DMA,
        ],
    )
    def agg_kernel(q_hbm, src_hbm, dst_hbm, out_hbm,
                   src_v, dst_v, rows_v, acc_sh, sem):
        c = lax.axis_index("c")
        s = lax.axis_index("s")
        wid = s * NC + c

        def zfill(r, carry):
            for cc in range(d // LANES):
                rows_v[r, pl.ds(cc * LANES, LANES)] = jnp.zeros(
                    (LANES,), jnp.float32)
            return carry

        lax.fori_loop(0, CHUNK, zfill, 0)
        for t in range(stripe // CHUNK):
            pltpu.sync_copy(rows_v,
                            acc_sh.at[pl.ds(s * stripe + t * CHUNK, CHUNK)])
        plsc.subcore_barrier()

        pltpu.sync_copy(src_hbm.at[wid], src_v)
        pltpu.sync_copy(dst_hbm.at[wid], dst_v)

        def body(j, carry):
            pltpu.async_copy(q_hbm.at[src_v.at[j]], rows_v, sem).wait()
            pltpu.sync_copy(rows_v, acc_sh.at[dst_v.at[j]], add=True)
            return carry

        nch = jnp.where(c == 0, t0, t1)
        lax.fori_loop(0, nch, body, 0)
        plsc.subcore_barrier()
        pltpu.sync_copy(acc_sh.at[pl.ds(s * stripe, stripe)],
                        out_hbm.at[c, pl.ds(s * stripe, stripe)])

    return agg_kernel(q, src_r, dst_r)


def _tc_scale_matmul(dg0, dg1, x, w, rows):
    n_pad, d = x.shape
    grid = (n_pad // rows,)

    def body(dg0_ref, dg1_ref, x_ref, w_ref, q_ref):
        deg = dg0_ref[...] + dg1_ref[...]
        dinv = lax.rsqrt(jnp.maximum(deg, 1.0))
        p = jnp.dot(x_ref[...], w_ref[...],
                    preferred_element_type=jnp.float32)
        q_ref[...] = dinv * p

    return pl.pallas_call(
        body,
        grid=grid,
        in_specs=[
            pl.BlockSpec((rows, 1), lambda i: (i, 0)),
            pl.BlockSpec((rows, 1), lambda i: (i, 0)),
            pl.BlockSpec((rows, d), lambda i: (i, 0)),
            pl.BlockSpec((d, d), lambda i: (0, 0)),
        ],
        out_specs=pl.BlockSpec((rows, d), lambda i: (i, 0)),
        out_shape=jax.ShapeDtypeStruct((n_pad, d), jnp.float32),
    )(dg0, dg1, x, w)


def _tc_combine_matmul(dg0, dg1, t0, t1, b1, w, rows):
    n_pad, d = t0.shape
    grid = (n_pad // rows,)

    def body(dg0_ref, dg1_ref, t0_ref, t1_ref, b1_ref, w_ref, q_ref):
        deg = dg0_ref[...] + dg1_ref[...]
        dinv = lax.rsqrt(jnp.maximum(deg, 1.0))
        h = jnp.maximum(dinv * (t0_ref[...] + t1_ref[...]) + b1_ref[...], 0.0)
        p = jnp.dot(h, w_ref[...], preferred_element_type=jnp.float32)
        q_ref[...] = dinv * p

    return pl.pallas_call(
        body,
        grid=grid,
        in_specs=[
            pl.BlockSpec((rows, 1), lambda i: (i, 0)),
            pl.BlockSpec((rows, 1), lambda i: (i, 0)),
            pl.BlockSpec((rows, d), lambda i: (i, 0)),
            pl.BlockSpec((rows, d), lambda i: (i, 0)),
            pl.BlockSpec((1, d), lambda i: (0, 0)),
            pl.BlockSpec((d, d), lambda i: (0, 0)),
        ],
        out_specs=pl.BlockSpec((rows, d), lambda i: (i, 0)),
        out_shape=jax.ShapeDtypeStruct((n_pad, d), jnp.float32),
    )(dg0, dg1, t0, t1, b1, w)


def _tc_combine(dg0, dg1, t0, t1, b2, rows):
    n_pad, d = t0.shape
    grid = (n_pad // rows,)

    def body(dg0_ref, dg1_ref, t0_ref, t1_ref, b2_ref, o_ref):
        deg = dg0_ref[...] + dg1_ref[...]
        dinv = lax.rsqrt(jnp.maximum(deg, 1.0))
        o_ref[...] = dinv * (t0_ref[...] + t1_ref[...]) + b2_ref[...]

    return pl.pallas_call(
        body,
        grid=grid,
        in_specs=[
            pl.BlockSpec((rows, 1), lambda i: (i, 0)),
            pl.BlockSpec((rows, 1), lambda i: (i, 0)),
            pl.BlockSpec((rows, d), lambda i: (i, 0)),
            pl.BlockSpec((rows, d), lambda i: (i, 0)),
            pl.BlockSpec((1, d), lambda i: (0, 0)),
        ],
        out_specs=pl.BlockSpec((rows, d), lambda i: (i, 0)),
        out_shape=jax.ShapeDtypeStruct((n_pad, d), jnp.float32),
    )(dg0, dg1, t0, t1, b2)


def kernel(x, edge_index, W1, b1, W2, b2):
    n, d = x.shape
    e = edge_index.shape[1]

    stripe_unit = NS * CHUNK
    n_pad = ((n + 1 + stripe_unit - 1) // stripe_unit) * stripe_unit
    rows = 1024 if n_pad % 1024 == 0 else CHUNK

    loop = jnp.arange(n, dtype=edge_index.dtype)
    s_ext = jnp.concatenate([edge_index[0], loop])
    d_ext = jnp.concatenate([edge_index[1], loop])
    e_ext = e + n
    cpw = (e_ext + NW * CHUNK - 1) // (NW * CHUNK)
    e_pad = NW * cpw * CHUNK
    s_ext = jnp.pad(s_ext, (0, e_pad - e_ext))
    d_ext = jnp.pad(d_ext, (0, e_pad - e_ext), constant_values=n)

    t0 = (2 * cpw * 70 + 81) // 162
    t1 = 2 * cpw - t0
    counts = [t0 if (w % NC) == 0 else t1 for w in range(NW)]
    cpw_max = max(t0, t1)
    rows_s, rows_d = [], []
    off = 0
    for w in range(NW):
        cw = counts[w]
        blk_s = s_ext[off:off + cw * CHUNK].reshape(cw, CHUNK)
        blk_d = d_ext[off:off + cw * CHUNK].reshape(cw, CHUNK)
        if cw < cpw_max:
            blk_s = jnp.pad(blk_s, ((0, cpw_max - cw), (0, 0)))
            blk_d = jnp.pad(blk_d, ((0, cpw_max - cw), (0, 0)),
                            constant_values=n)
        rows_s.append(blk_s)
        rows_d.append(blk_d)
        off += cw * CHUNK
    src_r = jnp.stack(rows_s)
    dst_r = jnp.stack(rows_d)

    xp = jnp.pad(x, ((0, n_pad - n), (0, 0)))
    b1r = b1.reshape(1, d)
    b2r = b2.reshape(1, d)

    degp = _sc_degree(dst_r, n_pad)
    dg0 = degp[0].reshape(n_pad, 1)
    dg1 = degp[1].reshape(n_pad, 1)

    q1 = _tc_scale_matmul(dg0, dg1, xp, W1, rows)
    acc1 = _sc_aggregate(q1, src_r, dst_r, n_pad, t0, t1)
    q2 = _tc_combine_matmul(dg0, dg1, acc1[0], acc1[1], b1r, W2, rows)
    acc2 = _sc_aggregate(q2, src_r, dst_r, n_pad, t0, t1)
    outp = _tc_combine(dg0, dg1, acc2[0], acc2[1], b2r, rows)
    return outp[:n]


# speedup vs baseline: 1.1547x; 1.1547x over previous
import functools

import jax
import jax.numpy as jnp
from jax import lax
from jax.experimental import pallas as pl
from jax.experimental.pallas import tpu as pltpu
from jax.experimental.pallas import tpu_sc as plsc

NC = 2
NS = 16
LANES = 16
NW = NC * NS
CHUNK = 128
DEG_W = 16


def _sc_degree(dst_r, n_pad):
    cpw = dst_r.shape[1]
    stripe = n_pad // NS
    mesh = plsc.VectorSubcoreMesh(core_axis_name="c", subcore_axis_name="s")

    @functools.partial(
        pl.kernel,
        out_type=jax.ShapeDtypeStruct((NC, n_pad), jnp.float32),
        mesh=mesh,
        scratch_types=[
            pltpu.VMEM((cpw, CHUNK), jnp.int32),
            pltpu.VMEM((CHUNK,), jnp.float32),
            pltpu.VMEM((stripe,), jnp.float32),
            pltpu.VMEM_SHARED((n_pad,), jnp.float32),
        ],
    )
    def deg_kernel(dst_hbm, out_hbm, dst_v, ones_v, zero_v, hist_sh):
        c = lax.axis_index("c")
        s = lax.axis_index("s")
        wid = s * NC + c

        for r in range(CHUNK // LANES):
            ones_v[pl.ds(r * LANES, LANES)] = jnp.ones((LANES,), jnp.float32)

        def zfill(r, carry):
            zero_v[pl.ds(r * LANES, LANES)] = jnp.zeros((LANES,), jnp.float32)
            return carry

        lax.fori_loop(0, stripe // LANES, zfill, 0)
        pltpu.sync_copy(zero_v, hist_sh.at[pl.ds(s * stripe, stripe)])
        plsc.subcore_barrier()

        pltpu.sync_copy(dst_hbm.at[wid], dst_v)

        def body(j, carry):
            pltpu.sync_copy(ones_v, hist_sh.at[dst_v.at[j]], add=True)
            return carry

        lax.fori_loop(0, cpw, body, 0)
        plsc.subcore_barrier()
        pltpu.sync_copy(hist_sh.at[pl.ds(s * stripe, stripe)],
                        out_hbm.at[c, pl.ds(s * stripe, stripe)])

    return deg_kernel(dst_r)


def _sc_aggregate(q, src_r, dst_r, n_pad, t0, t1):
    d = q.shape[1]
    cpw = src_r.shape[1]
    assert max(t0, t1) <= cpw
    stripe = n_pad // NS
    mesh = plsc.VectorSubcoreMesh(core_axis_name="c", subcore_axis_name="s")

    @functools.partial(
        pl.kernel,
        out_type=jax.ShapeDtypeStruct((NC, n_pad, d), jnp.float32),
        mesh=mesh,
        scratch_types=[
            pltpu.VMEM((cpw, CHUNK), jnp.int32),
            pltpu.VMEM((cpw, CHUNK), jnp.int32),
            pltpu.VMEM((CHUNK, d), jnp.float32),
            pltpu.VMEM_SHARED((n_pad, d), jnp.float32),
            pltpu.SemaphoreType.DMA,
        ],
    )
    def agg_kernel(q_hbm, src_hbm, dst_hbm, out_hbm,
                   src_v, dst_v, rows_v, acc_sh, sem):
        c = lax.axis_index("c")
        s = lax.axis_index("s")
        wid = s * NC + c

        def zfill(r, carry):
            for cc in range(d // LANES):
                rows_v[r, pl.ds(cc * LANES, LANES)] = jnp.zeros(
                    (LANES,), jnp.float32)
            return carry

        lax.fori_loop(0, CHUNK, zfill, 0)
        for t in range(stripe // CHUNK):
            pltpu.sync_copy(rows_v,
                            acc_sh.at[pl.ds(s * stripe + t * CHUNK, CHUNK)])
        plsc.subcore_barrier()

        pltpu.sync_copy(src_hbm.at[wid], src_v)
        pltpu.sync_copy(dst_hbm.at[wid], dst_v)

        def body(j, carry):
            pltpu.async_copy(q_hbm.at[src_v.at[j]], rows_v, sem).wait()
            pltpu.sync_copy(rows_v, acc_sh.at[dst_v.at[j]], add=True)
            return carry

        nch = jnp.where(c == 0, t0, t1)
        lax.fori_loop(0, nch, body, 0)
        plsc.subcore_barrier()
        pltpu.sync_copy(acc_sh.at[pl.ds(s * stripe, stripe)],
                        out_hbm.at[c, pl.ds(s * stripe, stripe)])

    return agg_kernel(q, src_r, dst_r)


def _tc_scale_matmul(dg0, dg1, x, w, rows):
    n_pad, d = x.shape
    grid = (n_pad // rows,)

    def body(dg0_ref, dg1_ref, x_ref, w_ref, q_ref):
        deg = dg0_ref[...] + dg1_ref[...]
        dinv = lax.rsqrt(jnp.maximum(deg, 1.0))
        p = jnp.dot(x_ref[...], w_ref[...],
                    preferred_element_type=jnp.float32)
        q_ref[...] = dinv * p

    return pl.pallas_call(
        body,
        grid=grid,
        in_specs=[
            pl.BlockSpec((rows, 1), lambda i: (i, 0)),
            pl.BlockSpec((rows, 1), lambda i: (i, 0)),
            pl.BlockSpec((rows, d), lambda i: (i, 0)),
            pl.BlockSpec((d, d), lambda i: (0, 0)),
        ],
        out_specs=pl.BlockSpec((rows, d), lambda i: (i, 0)),
        out_shape=jax.ShapeDtypeStruct((n_pad, d), jnp.float32),
    )(dg0, dg1, x, w)


def _tc_combine_matmul(dg0, dg1, t0, t1, b1, w, rows):
    n_pad, d = t0.shape
    grid = (n_pad // rows,)

    def body(dg0_ref, dg1_ref, t0_ref, t1_ref, b1_ref, w_ref, q_ref):
        deg = dg0_ref[...] + dg1_ref[...]
        dinv = lax.rsqrt(jnp.maximum(deg, 1.0))
        h = jnp.maximum(dinv * (t0_ref[...] + t1_ref[...]) + b1_ref[...], 0.0)
        p = jnp.dot(h, w_ref[...], preferred_element_type=jnp.float32)
        q_ref[...] = dinv * p

    return pl.pallas_call(
        body,
        grid=grid,
        in_specs=[
            pl.BlockSpec((rows, 1), lambda i: (i, 0)),
            pl.BlockSpec((rows, 1), lambda i: (i, 0)),
            pl.BlockSpec((rows, d), lambda i: (i, 0)),
            pl.BlockSpec((rows, d), lambda i: (i, 0)),
            pl.BlockSpec((1, d), lambda i: (0, 0)),
            pl.BlockSpec((d, d), lambda i: (0, 0)),
        ],
        out_specs=pl.BlockSpec((rows, d), lambda i: (i, 0)),
        out_shape=jax.ShapeDtypeStruct((n_pad, d), jnp.float32),
    )(dg0, dg1, t0, t1, b1, w)


def _tc_combine(dg0, dg1, t0, t1, b2, rows):
    n_pad, d = t0.shape
    grid = (n_pad // rows,)

    def body(dg0_ref, dg1_ref, t0_ref, t1_ref, b2_ref, o_ref):
        deg = dg0_ref[...] + dg1_ref[...]
        dinv = lax.rsqrt(jnp.maximum(deg, 1.0))
        o_ref[...] = dinv * (t0_ref[...] + t1_ref[...]) + b2_ref[...]

    return pl.pallas_call(
        body,
        grid=grid,
        in_specs=[
            pl.BlockSpec((rows, 1), lambda i: (i, 0)),
            pl.BlockSpec((rows, 1), lambda i: (i, 0)),
            pl.BlockSpec((rows, d), lambda i: (i, 0)),
            pl.BlockSpec((rows, d), lambda i: (i, 0)),
            pl.BlockSpec((1, d), lambda i: (0, 0)),
        ],
        out_specs=pl.BlockSpec((rows, d), lambda i: (i, 0)),
        out_shape=jax.ShapeDtypeStruct((n_pad, d), jnp.float32),
    )(dg0, dg1, t0, t1, b2)


def kernel(x, edge_index, W1, b1, W2, b2):
    n, d = x.shape
    e = edge_index.shape[1]

    stripe_unit = NS * CHUNK
    n_pad = ((n + 1 + stripe_unit - 1) // stripe_unit) * stripe_unit
    rows = 1024 if n_pad % 1024 == 0 else CHUNK

    loop = jnp.arange(n, dtype=edge_index.dtype)
    s_ext = jnp.concatenate([edge_index[0], loop])
    d_ext = jnp.concatenate([edge_index[1], loop])
    e_ext = e + n
    cpw = (e_ext + NW * CHUNK - 1) // (NW * CHUNK)
    e_pad = NW * cpw * CHUNK
    s_ext = jnp.pad(s_ext, (0, e_pad - e_ext))
    d_ext = jnp.pad(d_ext, (0, e_pad - e_ext), constant_values=n)

    t1 = (2 * cpw * 70 + 81) // 162
    t0 = 2 * cpw - t1
    counts = [t0 if (w % NC) == 0 else t1 for w in range(NW)]
    cpw_max = max(t0, t1)
    rows_s, rows_d = [], []
    off = 0
    for w in range(NW):
        cw = counts[w]
        blk_s = s_ext[off:off + cw * CHUNK].reshape(cw, CHUNK)
        blk_d = d_ext[off:off + cw * CHUNK].reshape(cw, CHUNK)
        if cw < cpw_max:
            blk_s = jnp.pad(blk_s, ((0, cpw_max - cw), (0, 0)))
            blk_d = jnp.pad(blk_d, ((0, cpw_max - cw), (0, 0)),
                            constant_values=n)
        rows_s.append(blk_s)
        rows_d.append(blk_d)
        off += cw * CHUNK
    src_r = jnp.stack(rows_s)
    dst_r = jnp.stack(rows_d)

    xp = jnp.pad(x, ((0, n_pad - n), (0, 0)))
    b1r = b1.reshape(1, d)
    b2r = b2.reshape(1, d)

    degp = _sc_degree(dst_r, n_pad)
    dg0 = degp[0].reshape(n_pad, 1)
    dg1 = degp[1].reshape(n_pad, 1)

    q1 = _tc_scale_matmul(dg0, dg1, xp, W1, rows)
    acc1 = _sc_aggregate(q1, src_r, dst_r, n_pad, t0, t1)
    q2 = _tc_combine_matmul(dg0, dg1, acc1[0], acc1[1], b1r, W2, rows)
    acc2 = _sc_aggregate(q2, src_r, dst_r, n_pad, t0, t1)
    outp = _tc_combine(dg0, dg1, acc2[0], acc2[1], b2r, rows)
    return outp[:n]


# speedup vs baseline: 1.1774x; 1.0196x over previous
import functools

import jax
import jax.numpy as jnp
from jax import lax
from jax.experimental import pallas as pl
from jax.experimental.pallas import tpu as pltpu
from jax.experimental.pallas import tpu_sc as plsc

NC = 2
NS = 16
LANES = 16
NW = NC * NS
CHUNK = 128
DEG_W = 16


def _sc_degree(dst_r, n_pad):
    cpw = dst_r.shape[1]
    stripe = n_pad // NS
    mesh = plsc.VectorSubcoreMesh(core_axis_name="c", subcore_axis_name="s")

    @functools.partial(
        pl.kernel,
        out_type=jax.ShapeDtypeStruct((NC, n_pad), jnp.float32),
        mesh=mesh,
        scratch_types=[
            pltpu.VMEM((cpw, CHUNK), jnp.int32),
            pltpu.VMEM((CHUNK,), jnp.float32),
            pltpu.VMEM((stripe,), jnp.float32),
            pltpu.VMEM_SHARED((n_pad,), jnp.float32),
        ],
    )
    def deg_kernel(dst_hbm, out_hbm, dst_v, ones_v, zero_v, hist_sh):
        c = lax.axis_index("c")
        s = lax.axis_index("s")
        wid = s * NC + c

        for r in range(CHUNK // LANES):
            ones_v[pl.ds(r * LANES, LANES)] = jnp.ones((LANES,), jnp.float32)

        def zfill(r, carry):
            zero_v[pl.ds(r * LANES, LANES)] = jnp.zeros((LANES,), jnp.float32)
            return carry

        lax.fori_loop(0, stripe // LANES, zfill, 0)
        pltpu.sync_copy(zero_v, hist_sh.at[pl.ds(s * stripe, stripe)])
        plsc.subcore_barrier()

        pltpu.sync_copy(dst_hbm.at[wid], dst_v)

        def body(j, carry):
            pltpu.sync_copy(ones_v, hist_sh.at[dst_v.at[j]], add=True)
            return carry

        lax.fori_loop(0, cpw, body, 0)
        plsc.subcore_barrier()
        pltpu.sync_copy(hist_sh.at[pl.ds(s * stripe, stripe)],
                        out_hbm.at[c, pl.ds(s * stripe, stripe)])

    return deg_kernel(dst_r)


def _sc_aggregate(q, src_r, dst_r, n_pad):
    d = q.shape[1]
    cpw = src_r.shape[1]
    stripe = n_pad // NS
    mesh = plsc.VectorSubcoreMesh(core_axis_name="c", subcore_axis_name="s")

    @functools.partial(
        pl.kernel,
        out_type=jax.ShapeDtypeStruct((NC, n_pad, d), jnp.float32),
        mesh=mesh,
        scratch_types=[
            pltpu.VMEM((cpw, CHUNK), jnp.int32),
            pltpu.VMEM((cpw, CHUNK), jnp.int32),
            pltpu.VMEM((CHUNK, d), jnp.float32),
            pltpu.VMEM_SHARED((n_pad, d), jnp.float32),
            pltpu.SemaphoreType.DMA,
        ],
    )
    def agg_kernel(q_hbm, src_hbm, dst_hbm, out_hbm,
                   src_v, dst_v, rows_v, acc_sh, sem):
        c = lax.axis_index("c")
        s = lax.axis_index("s")
        wid = s * NC + c

        def zfill(r, carry):
            for cc in range(d // LANES):
                rows_v[r, pl.ds(cc * LANES, LANES)] = jnp.zeros(
                    (LANES,), jnp.float32)
            return carry

        lax.fori_loop(0, CHUNK, zfill, 0)
        for t in range(stripe // CHUNK):
            pltpu.sync_copy(rows_v,
                            acc_sh.at[pl.ds(s * stripe + t * CHUNK, CHUNK)])
        plsc.subcore_barrier()

        pltpu.sync_copy(src_hbm.at[wid], src_v)
        pltpu.sync_copy(dst_hbm.at[wid], dst_v)

        def body(j, carry):
            pltpu.async_copy(q_hbm.at[src_v.at[j]], rows_v, sem).wait()
            pltpu.sync_copy(rows_v, acc_sh.at[dst_v.at[j]], add=True)
            return carry

        lax.fori_loop(0, cpw, body, 0)
        plsc.subcore_barrier()
        pltpu.sync_copy(acc_sh.at[pl.ds(s * stripe, stripe)],
                        out_hbm.at[c, pl.ds(s * stripe, stripe)])

    return agg_kernel(q, src_r, dst_r)


def _tc_scale_matmul(dg0, dg1, x, w, rows):
    n_pad, d = x.shape
    grid = (n_pad // rows,)

    def body(dg0_ref, dg1_ref, x_ref, w_ref, q_ref):
        deg = dg0_ref[...] + dg1_ref[...]
        dinv = lax.rsqrt(jnp.maximum(deg, 1.0))
        p = jnp.dot(x_ref[...], w_ref[...],
                    preferred_element_type=jnp.float32)
        q_ref[...] = dinv * p

    return pl.pallas_call(
        body,
        grid=grid,
        in_specs=[
            pl.BlockSpec((rows, 1), lambda i: (i, 0)),
            pl.BlockSpec((rows, 1), lambda i: (i, 0)),
            pl.BlockSpec((rows, d), lambda i: (i, 0)),
            pl.BlockSpec((d, d), lambda i: (0, 0)),
        ],
        out_specs=pl.BlockSpec((rows, d), lambda i: (i, 0)),
        out_shape=jax.ShapeDtypeStruct((n_pad, d), jnp.float32),
    )(dg0, dg1, x, w)


def _tc_combine_matmul(dg0, dg1, t0, t1, b1, w, rows):
    n_pad, d = t0.shape
    grid = (n_pad // rows,)

    def body(dg0_ref, dg1_ref, t0_ref, t1_ref, b1_ref, w_ref, q_ref):
        deg = dg0_ref[...] + dg1_ref[...]
        dinv = lax.rsqrt(jnp.maximum(deg, 1.0))
        h = jnp.maximum(dinv * (t0_ref[...] + t1_ref[...]) + b1_ref[...], 0.0)
        p = jnp.dot(h, w_ref[...], preferred_element_type=jnp.float32)
        q_ref[...] = dinv * p

    return pl.pallas_call(
        body,
        grid=grid,
        in_specs=[
            pl.BlockSpec((rows, 1), lambda i: (i, 0)),
            pl.BlockSpec((rows, 1), lambda i: (i, 0)),
            pl.BlockSpec((rows, d), lambda i: (i, 0)),
            pl.BlockSpec((rows, d), lambda i: (i, 0)),
            pl.BlockSpec((1, d), lambda i: (0, 0)),
            pl.BlockSpec((d, d), lambda i: (0, 0)),
        ],
        out_specs=pl.BlockSpec((rows, d), lambda i: (i, 0)),
        out_shape=jax.ShapeDtypeStruct((n_pad, d), jnp.float32),
    )(dg0, dg1, t0, t1, b1, w)


def _tc_combine(dg0, dg1, t0, t1, b2, n_out):
    _, d = t0.shape
    rows = next(r for r in (1024, 1000, 800, 640, 625, 500, 400, 250, 200,
                            128, 125, 100, 80, 64, 50, 40, 32, 25, 20, 16,
                            10, 8, 5, 4, 2, 1) if n_out % r == 0)
    grid = (n_out // rows,)

    def body(dg0_ref, dg1_ref, t0_ref, t1_ref, b2_ref, o_ref):
        deg = dg0_ref[...] + dg1_ref[...]
        dinv = lax.rsqrt(jnp.maximum(deg, 1.0))
        o_ref[...] = dinv * (t0_ref[...] + t1_ref[...]) + b2_ref[...]

    return pl.pallas_call(
        body,
        grid=grid,
        in_specs=[
            pl.BlockSpec((rows, 1), lambda i: (i, 0)),
            pl.BlockSpec((rows, 1), lambda i: (i, 0)),
            pl.BlockSpec((rows, d), lambda i: (i, 0)),
            pl.BlockSpec((rows, d), lambda i: (i, 0)),
            pl.BlockSpec((1, d), lambda i: (0, 0)),
        ],
        out_specs=pl.BlockSpec((rows, d), lambda i: (i, 0)),
        out_shape=jax.ShapeDtypeStruct((n_out, d), jnp.float32),
    )(dg0, dg1, t0, t1, b2)


def kernel(x, edge_index, W1, b1, W2, b2):
    n, d = x.shape
    e = edge_index.shape[1]

    stripe_unit = NS * CHUNK
    n_pad = ((n + 1 + stripe_unit - 1) // stripe_unit) * stripe_unit
    rows = 1024 if n_pad % 1024 == 0 else CHUNK

    loop = jnp.arange(n, dtype=edge_index.dtype)
    s_ext = jnp.concatenate([edge_index[0], loop])
    d_ext = jnp.concatenate([edge_index[1], loop])
    e_ext = e + n
    cpw = (e_ext + NW * CHUNK - 1) // (NW * CHUNK)
    e_pad = NW * cpw * CHUNK
    s_ext = jnp.pad(s_ext, (0, e_pad - e_ext))
    d_ext = jnp.pad(d_ext, (0, e_pad - e_ext), constant_values=n)
    src_r = s_ext.reshape(NW, cpw, CHUNK)
    dst_r = d_ext.reshape(NW, cpw, CHUNK)

    xp = jnp.pad(x, ((0, n_pad - n), (0, 0)))
    b1r = b1.reshape(1, d)
    b2r = b2.reshape(1, d)

    degp = _sc_degree(dst_r, n_pad)
    dg0 = degp[0].reshape(n_pad, 1)
    dg1 = degp[1].reshape(n_pad, 1)

    q1 = _tc_scale_matmul(dg0, dg1, xp, W1, rows)
    acc1 = _sc_aggregate(q1, src_r, dst_r, n_pad)
    q2 = _tc_combine_matmul(dg0, dg1, acc1[0], acc1[1], b1r, W2, rows)
    acc2 = _sc_aggregate(q2, src_r, dst_r, n_pad)
    return _tc_combine(dg0, dg1, acc2[0], acc2[1], b2r, n)


# speedup vs baseline: 1.1790x; 1.0013x over previous
import functools

import jax
import jax.numpy as jnp
from jax import lax
from jax.experimental import pallas as pl
from jax.experimental.pallas import tpu as pltpu
from jax.experimental.pallas import tpu_sc as plsc

NC = 2
NS = 16
LANES = 16
NW = NC * NS
CHUNK = 128


def _sc_degree(dst_r, n_pad):
    cpw = dst_r.shape[1]
    stripe = n_pad // NS
    mesh = plsc.VectorSubcoreMesh(core_axis_name="c", subcore_axis_name="s")

    @functools.partial(
        pl.kernel,
        out_type=jax.ShapeDtypeStruct((NC, n_pad), jnp.float32),
        mesh=mesh,
        scratch_types=[
            pltpu.VMEM((cpw, CHUNK), jnp.int32),
            pltpu.VMEM((CHUNK,), jnp.float32),
            pltpu.VMEM((stripe,), jnp.float32),
            pltpu.VMEM_SHARED((n_pad,), jnp.float32),
        ],
    )
    def deg_kernel(dst_hbm, out_hbm, dst_v, ones_v, zero_v, hist_sh):
        c = lax.axis_index("c")
        s = lax.axis_index("s")
        wid = s * NC + c

        for r in range(CHUNK // LANES):
            ones_v[pl.ds(r * LANES, LANES)] = jnp.ones((LANES,), jnp.float32)

        def zfill(r, carry):
            zero_v[pl.ds(r * LANES, LANES)] = jnp.zeros((LANES,), jnp.float32)
            return carry

        lax.fori_loop(0, stripe // LANES, zfill, 0)
        pltpu.sync_copy(zero_v, hist_sh.at[pl.ds(s * stripe, stripe)])
        plsc.subcore_barrier()

        pltpu.sync_copy(dst_hbm.at[wid], dst_v)

        def body(j, carry):
            pltpu.sync_copy(ones_v, hist_sh.at[dst_v.at[j]], add=True)
            return carry

        lax.fori_loop(0, cpw, body, 0)
        plsc.subcore_barrier()
        pltpu.sync_copy(hist_sh.at[pl.ds(s * stripe, stripe)],
                        out_hbm.at[c, pl.ds(s * stripe, stripe)])

    return deg_kernel(dst_r)


def _sc_aggregate(q, src_r, dst_r, n_pad):
    d = q.shape[1]
    cpw = src_r.shape[1]
    stripe = n_pad // NS
    mesh = plsc.VectorSubcoreMesh(core_axis_name="c", subcore_axis_name="s")

    @functools.partial(
        pl.kernel,
        out_type=jax.ShapeDtypeStruct((NC, n_pad, d), jnp.float32),
        mesh=mesh,
        scratch_types=[
            pltpu.VMEM((cpw, CHUNK), jnp.int32),
            pltpu.VMEM((cpw, CHUNK), jnp.int32),
            pltpu.VMEM((CHUNK, d), jnp.float32),
            pltpu.VMEM_SHARED((n_pad, d), jnp.float32),
            pltpu.SemaphoreType.DMA,
        ],
    )
    def agg_kernel(q_hbm, src_hbm, dst_hbm, out_hbm,
                   src_v, dst_v, rows_v, acc_sh, sem):
        c = lax.axis_index("c")
        s = lax.axis_index("s")
        wid = s * NC + c

        def zfill(r, carry):
            for cc in range(d // LANES):
                rows_v[r, pl.ds(cc * LANES, LANES)] = jnp.zeros(
                    (LANES,), jnp.float32)
            return carry

        lax.fori_loop(0, CHUNK, zfill, 0)
        for t in range(stripe // CHUNK):
            pltpu.sync_copy(rows_v,
                            acc_sh.at[pl.ds(s * stripe + t * CHUNK, CHUNK)])
        plsc.subcore_barrier()

        pltpu.sync_copy(src_hbm.at[wid], src_v)
        pltpu.sync_copy(dst_hbm.at[wid], dst_v)

        def body(j, carry):
            pltpu.async_copy(q_hbm.at[src_v.at[j]], rows_v, sem).wait()
            pltpu.sync_copy(rows_v, acc_sh.at[dst_v.at[j]], add=True)
            return carry

        lax.fori_loop(0, cpw, body, 0)
        plsc.subcore_barrier()
        pltpu.sync_copy(acc_sh.at[pl.ds(s * stripe, stripe)],
                        out_hbm.at[c, pl.ds(s * stripe, stripe)])

    return agg_kernel(q, src_r, dst_r)


def _tc_scale_matmul(dg0, dg1, x, w, rows):
    n_pad, d = x.shape
    grid = (n_pad // rows,)

    def body(dg0_ref, dg1_ref, x_ref, w_ref, q_ref):
        deg = dg0_ref[...] + dg1_ref[...]
        dinv = lax.rsqrt(jnp.maximum(deg, 1.0))
        p = jnp.dot(x_ref[...], w_ref[...],
                    preferred_element_type=jnp.float32)
        q_ref[...] = dinv * p

    return pl.pallas_call(
        body,
        grid=grid,
        in_specs=[
            pl.BlockSpec((rows, 1), lambda i: (i, 0)),
            pl.BlockSpec((rows, 1), lambda i: (i, 0)),
            pl.BlockSpec((rows, d), lambda i: (i, 0)),
            pl.BlockSpec((d, d), lambda i: (0, 0)),
        ],
        out_specs=pl.BlockSpec((rows, d), lambda i: (i, 0)),
        out_shape=jax.ShapeDtypeStruct((n_pad, d), jnp.float32),
    )(dg0, dg1, x, w)


def _tc_combine_matmul(dg0, dg1, t0, t1, b1, w, rows):
    n_pad, d = t0.shape
    grid = (n_pad // rows,)

    def body(dg0_ref, dg1_ref, t0_ref, t1_ref, b1_ref, w_ref, q_ref):
        deg = dg0_ref[...] + dg1_ref[...]
        dinv = lax.rsqrt(jnp.maximum(deg, 1.0))
        h = jnp.maximum(dinv * (t0_ref[...] + t1_ref[...]) + b1_ref[...], 0.0)
        p = jnp.dot(h, w_ref[...], preferred_element_type=jnp.float32)
        q_ref[...] = dinv * p

    return pl.pallas_call(
        body,
        grid=grid,
        in_specs=[
            pl.BlockSpec((rows, 1), lambda i: (i, 0)),
            pl.BlockSpec((rows, 1), lambda i: (i, 0)),
            pl.BlockSpec((rows, d), lambda i: (i, 0)),
            pl.BlockSpec((rows, d), lambda i: (i, 0)),
            pl.BlockSpec((1, d), lambda i: (0, 0)),
            pl.BlockSpec((d, d), lambda i: (0, 0)),
        ],
        out_specs=pl.BlockSpec((rows, d), lambda i: (i, 0)),
        out_shape=jax.ShapeDtypeStruct((n_pad, d), jnp.float32),
    )(dg0, dg1, t0, t1, b1, w)


def _tc_combine(dg0, dg1, t0, t1, b2, n_out):
    _, d = t0.shape
    rows = next(r for r in (1024, 1000, 800, 640, 625, 500, 400, 250, 200,
                            128, 125, 100, 80, 64, 50, 40, 32, 25, 20, 16,
                            10, 8, 5, 4, 2, 1) if n_out % r == 0)
    grid = (n_out // rows,)

    def body(dg0_ref, dg1_ref, t0_ref, t1_ref, b2_ref, o_ref):
        deg = dg0_ref[...] + dg1_ref[...]
        dinv = lax.rsqrt(jnp.maximum(deg, 1.0))
        o_ref[...] = dinv * (t0_ref[...] + t1_ref[...]) + b2_ref[...]

    return pl.pallas_call(
        body,
        grid=grid,
        in_specs=[
            pl.BlockSpec((rows, 1), lambda i: (i, 0)),
            pl.BlockSpec((rows, 1), lambda i: (i, 0)),
            pl.BlockSpec((rows, d), lambda i: (i, 0)),
            pl.BlockSpec((rows, d), lambda i: (i, 0)),
            pl.BlockSpec((1, d), lambda i: (0, 0)),
        ],
        out_specs=pl.BlockSpec((rows, d), lambda i: (i, 0)),
        out_shape=jax.ShapeDtypeStruct((n_out, d), jnp.float32),
    )(dg0, dg1, t0, t1, b2)


def kernel(x, edge_index, W1, b1, W2, b2):
    n, d = x.shape
    e = edge_index.shape[1]

    stripe_unit = NS * CHUNK
    n_pad = ((n + 1 + stripe_unit - 1) // stripe_unit) * stripe_unit
    rows = 1024 if n_pad % 1024 == 0 else CHUNK

    loop = jnp.arange(n, dtype=edge_index.dtype)
    s_ext = jnp.concatenate([edge_index[0], loop])
    d_ext = jnp.concatenate([edge_index[1], loop])
    e_ext = e + n
    cpw = (e_ext + NW * CHUNK - 1) // (NW * CHUNK)
    e_pad = NW * cpw * CHUNK
    s_ext = jnp.pad(s_ext, (0, e_pad - e_ext))
    d_ext = jnp.pad(d_ext, (0, e_pad - e_ext), constant_values=n)
    src_r = s_ext.reshape(NW, cpw, CHUNK)
    dst_r = d_ext.reshape(NW, cpw, CHUNK)

    xp = jnp.pad(x, ((0, n_pad - n), (0, 0)))
    b1r = b1.reshape(1, d)
    b2r = b2.reshape(1, d)

    degp = _sc_degree(dst_r, n_pad)
    dg0 = degp[0].reshape(n_pad, 1)
    dg1 = degp[1].reshape(n_pad, 1)

    q1 = _tc_scale_matmul(dg0, dg1, xp, W1, rows)
    acc1 = _sc_aggregate(q1, src_r, dst_r, n_pad)
    q2 = _tc_combine_matmul(dg0, dg1, acc1[0], acc1[1], b1r, W2, rows)
    acc2 = _sc_aggregate(q2, src_r, dst_r, n_pad)
    return _tc_combine(dg0, dg1, acc2[0], acc2[1], b2r, n)
